# baseline (speedup 1.0000x reference)
import functools

import jax
import jax.numpy as jnp
from jax import lax
from jax.experimental import pallas as pl
from jax.experimental.pallas import tpu as pltpu
from jax.experimental.pallas import tpu_sc as plsc

B, N, D_AUDIO, D_TEXT, VOCAB = 4, 8, 768, 512, 49408
BN = B * N
V_BLK = 6176
N_BLK = VOCAB // V_BLK // 2


def _merge(qn, block, base, max_s, arg_s):
    norm2 = jnp.sum(block * block, axis=1)
    inv = 1.0 / jnp.maximum(jnp.sqrt(norm2), 1e-8)
    scores = lax.dot_general(qn, block,
                             (((1,), (1,)), ((), ())),
                             preferred_element_type=jnp.float32)
    scores = scores * inv[None, :]
    bmax = jnp.max(scores, axis=1)
    barg = jnp.argmax(scores, axis=1).astype(jnp.int32)
    improved = bmax > max_s[...]
    max_s[...] = jnp.where(improved, bmax, max_s[...])
    arg_s[...] = jnp.where(improved, base + barg, arg_s[...])


def _score_body(kw_ref, w_ref, b_ref, tab_a, tab_b, idx_ref, proj_ref,
                qn_s, max_s, arg_s):
    pid = pl.program_id(0)

    @pl.when(pid == 0)
    def _prologue():
        proj = jnp.dot(kw_ref[...], w_ref[...],
                       preferred_element_type=jnp.float32) + b_ref[...][None, :]
        proj_ref[...] = proj
        nrm = jnp.sqrt(jnp.sum(proj * proj, axis=1, keepdims=True))
        qn_s[...] = proj / jnp.maximum(nrm, 1e-8)
        max_s[...] = jnp.full((BN,), -jnp.inf, dtype=jnp.float32)
        arg_s[...] = jnp.zeros((BN,), dtype=jnp.int32)

    qn = qn_s[...]
    _merge(qn, tab_a[...], pid * V_BLK, max_s, arg_s)
    _merge(qn, tab_b[...], (N_BLK + pid) * V_BLK, max_s, arg_s)

    @pl.when(pid == N_BLK - 1)
    def _epilogue():
        idx_ref[...] = arg_s[...]


def _tc_scores(kw, w, b, table):
    return pl.pallas_call(
        _score_body,
        grid=(N_BLK,),
        in_specs=[
            pl.BlockSpec((BN, D_AUDIO), lambda i: (0, 0)),
            pl.BlockSpec((D_AUDIO, D_TEXT), lambda i: (0, 0)),
            pl.BlockSpec((D_TEXT,), lambda i: (0,)),
            pl.BlockSpec((V_BLK, D_TEXT), lambda i: (i, 0)),
            pl.BlockSpec((V_BLK, D_TEXT), lambda i: (N_BLK + i, 0)),
        ],
        out_specs=[
            pl.BlockSpec((BN,), lambda i: (0,)),
            pl.BlockSpec((BN, D_TEXT), lambda i: (0, 0)),
        ],
        out_shape=[
            jax.ShapeDtypeStruct((BN,), jnp.int32),
            jax.ShapeDtypeStruct((BN, D_TEXT), jnp.float32),
        ],
        scratch_shapes=[
            pltpu.VMEM((BN, D_TEXT), jnp.float32),
            pltpu.VMEM((BN,), jnp.float32),
            pltpu.VMEM((BN,), jnp.int32),
        ],
        compiler_params=pltpu.CompilerParams(
            dimension_semantics=("arbitrary",),
        ),
    )(kw, w, b, table, table)


ROWS_PER_W = 8
N_WORKERS = BN // ROWS_PER_W


def _sc_body(tab_hbm, idx_hbm, proj_hbm, out_hbm, idx_v, rows_v, proj_v,
             out_v, sem):
    wid = lax.axis_index("s") * 2 + lax.axis_index("c")

    @pl.when(wid < N_WORKERS)
    def _work():
        base = wid * ROWS_PER_W
        pltpu.sync_copy(idx_hbm.at[pl.ds(base, ROWS_PER_W)], idx_v)
        pltpu.async_copy(tab_hbm.at[idx_v], rows_v, sem).wait()
        pltpu.sync_copy(proj_hbm.at[pl.ds(base, ROWS_PER_W)], proj_v)
        for r in range(ROWS_PER_W):
            for c in range(0, D_TEXT, 16):
                p = proj_v[r, pl.ds(c, 16)]
                q = rows_v[r, pl.ds(c, 16)]
                out_v[r, pl.ds(c, 16)] = p + (q - p)
        pltpu.sync_copy(out_v, out_hbm.at[pl.ds(base, ROWS_PER_W)])


@functools.cache
def _sc_gather():
    return pl.kernel(
        _sc_body,
        out_type=jax.ShapeDtypeStruct((BN, D_TEXT), jnp.float32),
        mesh=plsc.VectorSubcoreMesh(core_axis_name="c", subcore_axis_name="s"),
        scratch_types=[
            pltpu.VMEM((ROWS_PER_W,), jnp.int32),
            pltpu.VMEM((ROWS_PER_W, D_TEXT), jnp.float32),
            pltpu.VMEM((ROWS_PER_W, D_TEXT), jnp.float32),
            pltpu.VMEM((ROWS_PER_W, D_TEXT), jnp.float32),
            pltpu.SemaphoreType.DMA,
        ],
    )


def kernel(keywords, W, b, token_embedding):
    kw = keywords.reshape(BN, D_AUDIO)
    idx, proj = _tc_scores(kw, W, b, token_embedding)
    out = _sc_gather()(token_embedding, idx, proj)
    return out.reshape(B, N, D_TEXT)


# speedup vs baseline: 1.1164x; 1.1164x over previous
import functools

import jax
import jax.numpy as jnp
from jax import lax
from jax.experimental import pallas as pl
from jax.experimental.pallas import tpu as pltpu
from jax.experimental.pallas import tpu_sc as plsc

B, N, D_AUDIO, D_TEXT, VOCAB = 4, 8, 768, 512, 49408
BN = B * N
V_BLK = 6176
N_BLK = VOCAB // V_BLK


def _score_body(kw_ref, w_ref, b_ref, tab_ref, idx_ref, qn_s, max_s, arg_s):
    pid = pl.program_id(0)

    @pl.when(pid == 0)
    def _prologue():
        proj = jnp.dot(kw_ref[...], w_ref[...],
                       preferred_element_type=jnp.float32) + b_ref[...][None, :]
        nrm = jnp.sqrt(jnp.sum(proj * proj, axis=1, keepdims=True))
        qn_s[...] = proj / jnp.maximum(nrm, 1e-8)
        max_s[...] = jnp.full((BN,), -jnp.inf, dtype=jnp.float32)
        arg_s[...] = jnp.zeros((BN,), dtype=jnp.int32)

    block = tab_ref[...]
    norm2 = jnp.sum(block * block, axis=1)
    inv = 1.0 / jnp.maximum(jnp.sqrt(norm2), 1e-8)
    scores = lax.dot_general(qn_s[...], block,
                             (((1,), (1,)), ((), ())),
                             preferred_element_type=jnp.float32)
    scores = scores * inv[None, :]
    bmax = jnp.max(scores, axis=1)
    barg = jnp.argmax(scores, axis=1).astype(jnp.int32)
    improved = bmax > max_s[...]
    max_s[...] = jnp.where(improved, bmax, max_s[...])
    arg_s[...] = jnp.where(improved, pid * V_BLK + barg, arg_s[...])

    @pl.when(pid == N_BLK - 1)
    def _epilogue():
        idx_ref[...] = arg_s[...]


def _tc_scores(kw, w, b, table):
    return pl.pallas_call(
        _score_body,
        grid=(N_BLK,),
        in_specs=[
            pl.BlockSpec((BN, D_AUDIO), lambda i: (0, 0)),
            pl.BlockSpec((D_AUDIO, D_TEXT), lambda i: (0, 0)),
            pl.BlockSpec((D_TEXT,), lambda i: (0,)),
            pl.BlockSpec((V_BLK, D_TEXT), lambda i: (i, 0)),
        ],
        out_specs=pl.BlockSpec((BN,), lambda i: (0,)),
        out_shape=jax.ShapeDtypeStruct((BN,), jnp.int32),
        scratch_shapes=[
            pltpu.VMEM((BN, D_TEXT), jnp.float32),
            pltpu.VMEM((BN,), jnp.float32),
            pltpu.VMEM((BN,), jnp.int32),
        ],
        compiler_params=pltpu.CompilerParams(
            dimension_semantics=("arbitrary",),
        ),
    )(kw, w, b, table)


ROWS_PER_W = 8
N_WORKERS = BN // ROWS_PER_W


def _sc_body(tab_hbm, idx_hbm, out_hbm, idx_v, rows_v, sem):
    wid = lax.axis_index("s") * 2 + lax.axis_index("c")

    @pl.when(wid < N_WORKERS)
    def _work():
        base = wid * ROWS_PER_W
        pltpu.sync_copy(idx_hbm.at[pl.ds(base, ROWS_PER_W)], idx_v)
        pltpu.async_copy(tab_hbm.at[idx_v], rows_v, sem).wait()
        pltpu.sync_copy(rows_v, out_hbm.at[pl.ds(base, ROWS_PER_W)])


@functools.cache
def _sc_gather():
    return pl.kernel(
        _sc_body,
        out_type=jax.ShapeDtypeStruct((BN, D_TEXT), jnp.float32),
        mesh=plsc.VectorSubcoreMesh(core_axis_name="c", subcore_axis_name="s"),
        scratch_types=[
            pltpu.VMEM((ROWS_PER_W,), jnp.int32),
            pltpu.VMEM((ROWS_PER_W, D_TEXT), jnp.float32),
            pltpu.SemaphoreType.DMA,
        ],
    )


def kernel(keywords, W, b, token_embedding):
    kw = keywords.reshape(BN, D_AUDIO)
    idx = _tc_scores(kw, W, b, token_embedding)
    out = _sc_gather()(token_embedding, idx)
    return out.reshape(B, N, D_TEXT)
